# Initial kernel scaffold; baseline (speedup 1.0000x reference)
import jax
import jax.numpy as jnp
from jax.experimental import pallas as pl


def kernel(atom_list, bond_list, atom_degree_list, bond_degree_list, atom_mask, params):
    raise NotImplementedError("write your pallas kernel here")


# speedup vs baseline: 28.3652x; 28.3652x over previous
import functools

import jax
import jax.numpy as jnp
from jax.experimental import pallas as pl
from jax.experimental.pallas import tpu as pltpu

RADIUS = 2
T = 2
FDIM = 39
BDIM = 10
FP = 64
MOL = 128
NB = 6
BM = 8

PREC = jax.lax.Precision.DEFAULT


def _mm(x, w):
    return jax.lax.dot_general(
        x, w, (((1,), (0,)), ((), ())),
        preferred_element_type=jnp.float32, precision=PREC)


def _bmm(x, w):
    return jax.lax.dot_general(
        x, w, (((2,), (1,)), ((0,), (0,))),
        preferred_element_type=jnp.float32, precision=PREC)


def _leaky(x):
    return jnp.where(x >= 0, x, 0.01 * x)


def _elu(x):
    return jnp.where(x > 0, x, jnp.exp(jnp.minimum(x, 0.0)) - 1.0)


def _gru(x, h, wih_t, whh_t, bih, bhh):
    gi = _mm(x, wih_t) + bih
    gh = _mm(h, whh_t) + bhh
    r = jax.nn.sigmoid(gi[:, :FP] + gh[:, :FP])
    z = jax.nn.sigmoid(gi[:, FP:2 * FP] + gh[:, FP:2 * FP])
    n = jnp.tanh(gi[:, 2 * FP:] + r * gh[:, 2 * FP:])
    return (1.0 - z) * n + z * h


def _fused_kernel(a_ref, b_ref, ad_ref, bd_ref, am_ref,
                  aw_t, ab, na_t, nb_t, nfb,
                  wa0, alb0, proj0, projb0, gih0, ghh0, bih0, bhh0,
                  wa1, alb1, proj1, projb1, gih1, ghh1, bih1, bhh1,
                  m_wa, m_alb, m_proj, m_projb,
                  m_gih, m_ghh, m_bih, m_bhh, out_wt, out_b,
                  af_out, mp_out):
    A = a_ref[...].reshape(BM * MOL, FDIM)
    Bd = b_ref[...].reshape(BM * MOL, BDIM)
    ad = ad_ref[...]
    bd = bd_ref[...]
    mask = am_ref[...]

    h = _leaky(_mm(A, aw_t[...]) + ab[...])

    P_a = _mm(A, na_t[...]).reshape(BM, MOL, FP)
    P_b = _mm(Bd, nb_t[...]).reshape(BM, MOL, FP)
    P = jnp.concatenate([P_a, P_b], axis=1)

    iota = jax.lax.broadcasted_iota(jnp.int32, (BM, MOL, MOL), 2)

    oh_a = []
    nf0 = []
    for n in range(NB):
        oa = (ad[:, :, n, None] == iota).astype(jnp.float32)
        ob = (bd[:, :, n, None] == iota).astype(jnp.float32)
        oh_a.append(oa)
        g = _bmm(jnp.concatenate([oa, ob], axis=2), P)
        nf0.append(_leaky(g + nfb[...]))

    pad = ad == (MOL - 1)
    smask = jnp.where(pad, -9e8, 0.0).astype(jnp.float32)
    amask = jnp.where(pad, 0.0, 1.0).astype(jnp.float32)

    for d in range(RADIUS):
        if d == 0:
            wa, alb, proj, projb = wa0, alb0, proj0, projb0
            gih, ghh, bih, bhh = gih0, ghh0, bih0, bhh0
            nf_cat = jnp.concatenate(nf0, axis=1)
            pr = _mm(nf_cat.reshape(BM * NB * MOL, FP), proj[...]) + projb[...]
            pr = pr.reshape(BM, NB, MOL, FP + 1)
        else:
            wa, alb, proj, projb = wa1, alb1, proj1, projb1
            gih, ghh, bih, bhh = gih1, ghh1, bih1, bhh1
            act = jnp.maximum(h, 0.0)
            Y = (_mm(act, proj[...]) + projb[...]).reshape(BM, MOL, FP + 1)
            pr = jnp.stack([_bmm(oh_a[n], Y) for n in range(NB)], axis=1)

        cur = h if d == 0 else act
        c = _mm(cur, wa[...]).reshape(BM, MOL, 1)

        q = jnp.concatenate([pr[:, n, :, FP:] for n in range(NB)], axis=2)
        score = _leaky(c + q + alb[...]) + smask
        score = score - jnp.max(score, axis=2, keepdims=True)
        e = jnp.exp(score)
        aw = e / jnp.sum(e, axis=2, keepdims=True) * amask

        ctx = aw[:, :, 0, None] * pr[:, 0, :, :FP]
        for n in range(1, NB):
            ctx = ctx + aw[:, :, n, None] * pr[:, n, :, :FP]
        x = _elu(ctx).reshape(BM * MOL, FP)
        h = _gru(x, h, gih[...], ghh[...], bih[...], bhh[...])

    af_out[...] = h.reshape(BM, MOL, FP)

    act = jnp.maximum(h, 0.0)
    act3 = act.reshape(BM, MOL, FP)
    mask3 = mask[:, :, None]
    mol = jnp.sum(act3 * mask3, axis=1)
    act_mol = jnp.maximum(mol, 0.0)
    msmask = jnp.where(mask3 == 0.0, -9e8, 0.0)

    aftq = _mm(act, m_proj[...]) + m_projb[...]
    aftq = aftq.reshape(BM, MOL, FP + 1)
    aft = aftq[:, :, :FP]
    qm = aftq[:, :, FP:]

    for _ in range(T):
        cm = _mm(act_mol, m_wa[...]).reshape(BM, 1, 1)
        s = _leaky(cm + qm + m_alb[...]) + msmask
        s = s - jnp.max(s, axis=1, keepdims=True)
        e = jnp.exp(s)
        maw = e / jnp.sum(e, axis=1, keepdims=True) * mask3
        mc = _elu(jnp.sum(maw * aft, axis=1))
        mol = _gru(mc, mol, m_gih[...], m_ghh[...], m_bih[...], m_bhh[...])
        act_mol = jnp.maximum(mol, 0.0)

    mp_out[...] = _mm(mol, out_wt[...]) + out_b[...]


def kernel(atom_list, bond_list, atom_degree_list, bond_degree_list, atom_mask, params):
    p = params
    B = atom_list.shape[0]

    f32 = jnp.float32
    aw_t = p['atom_fc_w'].T
    ab = p['atom_fc_b'][None, :]
    na_t = p['neighbor_fc_w'][:, :FDIM].T
    nb_t = p['neighbor_fc_w'][:, FDIM:].T
    nfb = p['neighbor_fc_b'][None, :]

    def radius_params(d):
        wa = p['align_w'][d, 0, :FP][:, None]
        wb = p['align_w'][d, 0, FP:][:, None]
        alb = p['align_b'][d][None, :]
        proj = jnp.concatenate([p['attend_w'][d].T, wb], axis=1)
        projb = jnp.concatenate([p['attend_b'][d], jnp.zeros((1,), f32)])[None, :]
        return (wa, alb, proj, projb,
                p['gru_wih'][d].T, p['gru_whh'][d].T,
                p['gru_bih'][d][None, :], p['gru_bhh'][d][None, :])

    r0 = radius_params(0)
    r1 = radius_params(1)

    m_wa = p['mol_align_w'][0, :FP][:, None]
    m_wb = p['mol_align_w'][0, FP:][:, None]
    m_alb = p['mol_align_b'][None, :]
    m_proj = jnp.concatenate([p['mol_attend_w'].T, m_wb], axis=1)
    m_projb = jnp.concatenate([p['mol_attend_b'], jnp.zeros((1,), f32)])[None, :]
    m_gih = p['mol_gru_wih'].T
    m_ghh = p['mol_gru_whh'].T
    m_bih = p['mol_gru_bih'][None, :]
    m_bhh = p['mol_gru_bhh'][None, :]
    out_wt = p['out_w'].T
    out_b = p['out_b'][None, :]

    consts = (aw_t, ab, na_t, nb_t, nfb,
              *r0, *r1,
              m_wa, m_alb, m_proj, m_projb,
              m_gih, m_ghh, m_bih, m_bhh, out_wt, out_b)

    def const_spec(x):
        return pl.BlockSpec(x.shape, lambda i: (0,) * x.ndim)

    grid = (B // BM,)
    in_specs = [
        pl.BlockSpec((BM, MOL, FDIM), lambda i: (i, 0, 0)),
        pl.BlockSpec((BM, MOL, BDIM), lambda i: (i, 0, 0)),
        pl.BlockSpec((BM, MOL, NB), lambda i: (i, 0, 0)),
        pl.BlockSpec((BM, MOL, NB), lambda i: (i, 0, 0)),
        pl.BlockSpec((BM, MOL), lambda i: (i, 0)),
    ] + [const_spec(c) for c in consts]

    out_shapes = (
        jax.ShapeDtypeStruct((B, MOL, FP), f32),
        jax.ShapeDtypeStruct((B, 1), f32),
    )
    out_specs = (
        pl.BlockSpec((BM, MOL, FP), lambda i: (i, 0, 0)),
        pl.BlockSpec((BM, 1), lambda i: (i, 0)),
    )

    af, mp = pl.pallas_call(
        _fused_kernel,
        grid=grid,
        in_specs=in_specs,
        out_specs=out_specs,
        out_shape=out_shapes,
        compiler_params=pltpu.CompilerParams(
            dimension_semantics=("parallel",)),
    )(atom_list, bond_list,
      atom_degree_list.astype(jnp.int32), bond_degree_list.astype(jnp.int32),
      atom_mask, *consts)
    return (af, mp)


# speedup vs baseline: 28.6823x; 1.0112x over previous
import functools

import jax
import jax.numpy as jnp
from jax.experimental import pallas as pl
from jax.experimental.pallas import tpu as pltpu

RADIUS = 2
T = 2
FDIM = 39
BDIM = 10
FP = 64
MOL = 128
NB = 6
BM = 8

PREC = jax.lax.Precision.DEFAULT


def _mm(x, w):
    return jax.lax.dot_general(
        x, w, (((1,), (0,)), ((), ())),
        preferred_element_type=jnp.float32, precision=PREC)


def _bmm(x, w, prec=PREC):
    return jax.lax.dot_general(
        x, w, (((2,), (1,)), ((0,), (0,))),
        preferred_element_type=jnp.float32, precision=prec)


def _leaky(x):
    return jnp.where(x >= 0, x, 0.01 * x)


def _elu(x):
    return jnp.where(x > 0, x, jnp.exp(jnp.minimum(x, 0.0)) - 1.0)


def _gru(x, h, wih_t, whh_t, bih, bhh):
    gi = _mm(x.astype(jnp.bfloat16), wih_t) + bih
    gh = _mm(h.astype(jnp.bfloat16), whh_t) + bhh
    r = jax.nn.sigmoid(gi[:, :FP] + gh[:, :FP])
    z = jax.nn.sigmoid(gi[:, FP:2 * FP] + gh[:, FP:2 * FP])
    n = jnp.tanh(gi[:, 2 * FP:] + r * gh[:, 2 * FP:])
    return (1.0 - z) * n + z * h


def _fused_kernel(a_ref, b_ref, ad_ref, bd_ref, am_ref,
                  aw_t, ab, na_t, nb_t, nfb,
                  wa0, alb0, proj0, projb0, gih0, ghh0, bih0, bhh0,
                  wa1, alb1, proj1, projb1, gih1, ghh1, bih1, bhh1,
                  m_wa, m_alb, m_proj, m_projb,
                  m_gih, m_ghh, m_bih, m_bhh, out_wt, out_b,
                  af_out, mp_out):
    A = a_ref[...].reshape(BM * MOL, FDIM)
    Bd = b_ref[...].reshape(BM * MOL, BDIM)
    ad = ad_ref[...]
    bd = bd_ref[...]
    mask = am_ref[...]

    A = A.astype(jnp.bfloat16)
    Bd = Bd.astype(jnp.bfloat16)
    h = _leaky(_mm(A, aw_t[...]) + ab[...])

    bf16 = jnp.bfloat16
    P_a = _mm(A, na_t[...]).reshape(BM, MOL, FP)
    P_b = _mm(Bd, nb_t[...]).reshape(BM, MOL, FP)
    P = jnp.concatenate([P_a, P_b], axis=1).astype(bf16)

    iota = jax.lax.broadcasted_iota(jnp.int32, (BM, MOL, MOL), 2)

    oh_a = []
    nf0 = []
    for n in range(NB):
        oa = (ad[:, :, n, None] == iota).astype(bf16)
        ob = (bd[:, :, n, None] == iota).astype(bf16)
        oh_a.append(oa)
        g = _bmm(jnp.concatenate([oa, ob], axis=2), P)
        nf0.append(_leaky(g + nfb[...]))

    pad = ad == (MOL - 1)
    smask = jnp.where(pad, -9e8, 0.0).astype(jnp.float32)
    amask = jnp.where(pad, 0.0, 1.0).astype(jnp.float32)

    ones_rows = (jax.lax.broadcasted_iota(jnp.int32, (NB, NB * FP), 1) // FP
                 == jax.lax.broadcasted_iota(jnp.int32, (NB, NB * FP), 0)
                 ).astype(bf16)

    for d in range(RADIUS):
        if d == 0:
            wa, alb, proj, projb = wa0, alb0, proj0, projb0
            gih, ghh, bih, bhh = gih0, ghh0, bih0, bhh0
            nf_cat = jnp.concatenate(nf0, axis=1).astype(bf16)
            pr = _mm(nf_cat.reshape(BM * NB * MOL, FP), proj[...]) + projb[...]
            pr = pr.reshape(BM, NB, MOL, FP + 1)
        else:
            wa, alb, proj, projb = wa1, alb1, proj1, projb1
            gih, ghh, bih, bhh = gih1, ghh1, bih1, bhh1
            act = jnp.maximum(h, 0.0)
            Y = (_mm(act, proj[...]) + projb[...]).reshape(BM, MOL, FP + 1)
            Yb = Y.astype(jnp.bfloat16)
            pr = jnp.stack([_bmm(oh_a[n], Yb) for n in range(NB)], axis=1)

        cur = h if d == 0 else act
        c = _mm(cur.astype(bf16), wa[...]).reshape(BM, MOL, 1)

        q = jnp.concatenate([pr[:, n, :, FP:] for n in range(NB)], axis=2)
        score = _leaky(c + q + alb[...]) + smask
        score = score - jnp.max(score, axis=2, keepdims=True)
        e = jnp.exp(score)
        aw = e / jnp.sum(e, axis=2, keepdims=True) * amask

        awb = _mm(aw.reshape(BM * MOL, NB).astype(bf16), ones_rows)
        ctx = awb[:, :FP] * pr[:, 0, :, :FP].reshape(BM * MOL, FP)
        for n in range(1, NB):
            ctx = ctx + (awb[:, n * FP:(n + 1) * FP]
                         * pr[:, n, :, :FP].reshape(BM * MOL, FP))
        x = _elu(ctx)
        h = _gru(x, h, gih[...], ghh[...], bih[...], bhh[...])

    af_out[...] = h.reshape(BM, MOL, FP)

    act = jnp.maximum(h, 0.0)
    act3 = act.reshape(BM, MOL, FP)
    mask3 = mask[:, :, None]
    mol = jnp.sum(act3 * mask3, axis=1)
    act_mol = jnp.maximum(mol, 0.0)
    msmask = jnp.where(mask3 == 0.0, -9e8, 0.0)

    aftq = _mm(act.astype(jnp.bfloat16), m_proj[...]) + m_projb[...]
    aftq = aftq.reshape(BM, MOL, FP + 1)
    aft = aftq[:, :, :FP]
    qm = aftq[:, :, FP:]

    for _ in range(T):
        cm = _mm(act_mol.astype(jnp.bfloat16), m_wa[...]).reshape(BM, 1, 1)
        s = _leaky(cm + qm + m_alb[...]) + msmask
        s = s - jnp.max(s, axis=1, keepdims=True)
        e = jnp.exp(s)
        maw = e / jnp.sum(e, axis=1, keepdims=True) * mask3
        mc = _elu(jnp.sum(maw * aft, axis=1))
        mol = _gru(mc, mol, m_gih[...], m_ghh[...], m_bih[...], m_bhh[...])
        act_mol = jnp.maximum(mol, 0.0)

    mp_out[...] = _mm(mol.astype(jnp.bfloat16), out_wt[...]) + out_b[...]


def kernel(atom_list, bond_list, atom_degree_list, bond_degree_list, atom_mask, params):
    p = params
    B = atom_list.shape[0]

    f32 = jnp.float32
    bf16 = jnp.bfloat16
    aw_t = p['atom_fc_w'].T.astype(bf16)
    ab = p['atom_fc_b'][None, :]
    na_t = p['neighbor_fc_w'][:, :FDIM].T.astype(bf16)
    nb_t = p['neighbor_fc_w'][:, FDIM:].T.astype(bf16)
    nfb = p['neighbor_fc_b'][None, :]

    def radius_params(d):
        wa = p['align_w'][d, 0, :FP][:, None].astype(bf16)
        wb = p['align_w'][d, 0, FP:][:, None]
        alb = p['align_b'][d][None, :]
        proj = jnp.concatenate([p['attend_w'][d].T, wb],
                               axis=1).astype(bf16)
        projb = jnp.concatenate([p['attend_b'][d], jnp.zeros((1,), f32)])[None, :]
        return (wa, alb, proj, projb,
                p['gru_wih'][d].T.astype(bf16), p['gru_whh'][d].T.astype(bf16),
                p['gru_bih'][d][None, :], p['gru_bhh'][d][None, :])

    r0 = radius_params(0)
    r1 = radius_params(1)

    m_wa = p['mol_align_w'][0, :FP][:, None].astype(bf16)
    m_wb = p['mol_align_w'][0, FP:][:, None]
    m_alb = p['mol_align_b'][None, :]
    m_proj = jnp.concatenate([p['mol_attend_w'].T, m_wb], axis=1).astype(bf16)
    m_projb = jnp.concatenate([p['mol_attend_b'], jnp.zeros((1,), f32)])[None, :]
    m_gih = p['mol_gru_wih'].T.astype(bf16)
    m_ghh = p['mol_gru_whh'].T.astype(bf16)
    m_bih = p['mol_gru_bih'][None, :]
    m_bhh = p['mol_gru_bhh'][None, :]
    out_wt = p['out_w'].T.astype(bf16)
    out_b = p['out_b'][None, :]

    consts = (aw_t, ab, na_t, nb_t, nfb,
              *r0, *r1,
              m_wa, m_alb, m_proj, m_projb,
              m_gih, m_ghh, m_bih, m_bhh, out_wt, out_b)

    def const_spec(x):
        return pl.BlockSpec(x.shape, lambda i: (0,) * x.ndim)

    grid = (B // BM,)
    in_specs = [
        pl.BlockSpec((BM, MOL, FDIM), lambda i: (i, 0, 0)),
        pl.BlockSpec((BM, MOL, BDIM), lambda i: (i, 0, 0)),
        pl.BlockSpec((BM, MOL, NB), lambda i: (i, 0, 0)),
        pl.BlockSpec((BM, MOL, NB), lambda i: (i, 0, 0)),
        pl.BlockSpec((BM, MOL), lambda i: (i, 0)),
    ] + [const_spec(c) for c in consts]

    out_shapes = (
        jax.ShapeDtypeStruct((B, MOL, FP), f32),
        jax.ShapeDtypeStruct((B, 1), f32),
    )
    out_specs = (
        pl.BlockSpec((BM, MOL, FP), lambda i: (i, 0, 0)),
        pl.BlockSpec((BM, 1), lambda i: (i, 0)),
    )

    af, mp = pl.pallas_call(
        _fused_kernel,
        grid=grid,
        in_specs=in_specs,
        out_specs=out_specs,
        out_shape=out_shapes,
        compiler_params=pltpu.CompilerParams(
            dimension_semantics=("parallel",)),
    )(atom_list, bond_list,
      atom_degree_list.astype(jnp.int32), bond_degree_list.astype(jnp.int32),
      atom_mask, *consts)
    return (af, mp)
